# baseline (speedup 1.0000x reference)
import functools

import jax
import jax.numpy as jnp
from jax import lax
from jax.experimental import pallas as pl
from jax.experimental.pallas import tpu as pltpu
from jax.experimental.pallas import tpu_sc as plsc

NUM_EMB = 1_000_000
DIM = 64
BATCH = 16384
HIST = 50
PHIST = 56
B_TOTAL = BATCH * PHIST * 2
NUM_CORES = 2
NUM_SUBCORES = 16
NW = NUM_CORES * NUM_SUBCORES
B_PER_W = B_TOTAL // NW
CHUNK = 896
N_CHUNKS = B_PER_W // CHUNK
N_PAIRS = N_CHUNKS // 2

_mesh = plsc.VectorSubcoreMesh(core_axis_name="c", subcore_axis_name="s")


@functools.partial(
    pl.kernel,
    mesh=_mesh,
    out_type=jax.ShapeDtypeStruct((B_TOTAL, DIM), jnp.float32),
    scratch_types=[
        pltpu.VMEM((2, CHUNK), jnp.int32),
        pltpu.VMEM((2, CHUNK, DIM), jnp.float32),
        pltpu.SemaphoreType.DMA,
        pltpu.SemaphoreType.DMA,
        pltpu.SemaphoreType.DMA,
        pltpu.SemaphoreType.DMA,
    ],
    compiler_params=pltpu.CompilerParams(use_tc_tiling_on_sc=False),
)
def _gather_kernel(idx_hbm, table_hbm, out_hbm, idx2, rows2, gA, gB, wA, wB):
    wid = lax.axis_index("s") * NUM_CORES + lax.axis_index("c")
    base = wid * B_PER_W
    idxA, idxB = idx2.at[0], idx2.at[1]
    rowsA, rowsB = rows2.at[0], rows2.at[1]

    def fire_gather(idx_ref, rows_ref, off, sem):
        pltpu.sync_copy(idx_hbm.at[pl.ds(off, CHUNK)], idx_ref)
        return pltpu.async_copy(table_hbm.at[idx_ref], rows_ref, sem)

    def fire_write(rows_ref, off, sem):
        return pltpu.async_copy(rows_ref, out_hbm.at[pl.ds(off, CHUNK)], sem)

    def drain_gather(rows_ref, sem):
        pltpu.make_async_copy(table_hbm.at[pl.ds(0, CHUNK)], rows_ref, sem).wait()

    def drain_write(rows_ref, sem):
        pltpu.make_async_copy(rows_ref, out_hbm.at[pl.ds(base, CHUNK)], sem).wait()

    dA = fire_gather(idxA, rowsA, base, gA)
    dB = fire_gather(idxB, rowsB, base + CHUNK, gB)
    dA.wait()
    fire_write(rowsA, base, wA)
    dB.wait()
    fire_write(rowsB, base + CHUNK, wB)
    drain_write(rowsA, wA)
    fire_gather(idxA, rowsA, base + 2 * CHUNK, gA)

    def body(g, carry):
        offA = base + (2 * g) * CHUNK
        offB = offA + CHUNK
        drain_write(rowsB, wB)
        dBg = fire_gather(idxB, rowsB, offB, gB)
        drain_gather(rowsA, gA)
        fire_write(rowsA, offA, wA)
        dBg.wait()
        fire_write(rowsB, offB, wB)
        drain_write(rowsA, wA)
        fire_gather(idxA, rowsA, offA + 2 * CHUNK, gA)
        return carry

    lax.fori_loop(1, N_PAIRS - 1, body, 0)

    offA = base + (N_CHUNKS - 2) * CHUNK
    offB = offA + CHUNK
    drain_write(rowsB, wB)
    dBl = fire_gather(idxB, rowsB, offB, gB)
    drain_gather(rowsA, gA)
    fire_write(rowsA, offA, wA)
    dBl.wait()
    fire_write(rowsB, offB, wB)
    drain_write(rowsA, wA)
    drain_write(rowsB, wB)


def kernel(token_ids, emb):
    ids_p = jnp.pad(token_ids, ((0, 0), (0, PHIST - HIST)))
    ids_p = ids_p.reshape(-1).astype(jnp.int32) * 2
    ids2 = jnp.stack([ids_p, ids_p + 1], axis=-1).reshape(-1)
    emb_p = jnp.pad(emb, ((0, 0), (0, DIM))).reshape(2 * NUM_EMB, DIM)
    out = _gather_kernel(ids2, emb_p)
    return out.reshape(BATCH, PHIST, 2 * DIM)[:, :HIST, :DIM]


# speedup vs baseline: 5.4316x; 5.4316x over previous
import functools

import jax
import jax.numpy as jnp
from jax import lax
from jax.experimental import pallas as pl
from jax.experimental.pallas import tpu as pltpu
from jax.experimental.pallas import tpu_sc as plsc

NUM_EMB = 1_000_000
DIM = 64
BATCH = 16384
HIST = 50
B_TOTAL = BATCH * HIST
NUM_CORES = 2
NUM_SUBCORES = 16
NW = NUM_CORES * NUM_SUBCORES
B_PER_W = B_TOTAL // NW
CHUNK = 800
N_CHUNKS = B_PER_W // CHUNK
N_PAIRS = N_CHUNKS // 2

_mesh = plsc.VectorSubcoreMesh(core_axis_name="c", subcore_axis_name="s")


@functools.partial(
    pl.kernel,
    mesh=_mesh,
    out_type=jax.ShapeDtypeStruct((B_TOTAL, DIM), jnp.float32),
    scratch_types=[
        pltpu.VMEM((2, CHUNK), jnp.int32),
        pltpu.VMEM((2, CHUNK, DIM), jnp.float32),
        pltpu.SemaphoreType.DMA,
        pltpu.SemaphoreType.DMA,
        pltpu.SemaphoreType.DMA,
        pltpu.SemaphoreType.DMA,
    ],
    compiler_params=pltpu.CompilerParams(use_tc_tiling_on_sc=False),
)
def _gather_kernel(idx_hbm, table_hbm, out_hbm, idx2, rows2, gA, gB, wA, wB):
    wid = lax.axis_index("s") * NUM_CORES + lax.axis_index("c")
    base = wid * B_PER_W
    idxA, idxB = idx2.at[0], idx2.at[1]
    rowsA, rowsB = rows2.at[0], rows2.at[1]

    def fire_gather(idx_ref, rows_ref, off, sem):
        pltpu.sync_copy(idx_hbm.at[pl.ds(off, CHUNK)], idx_ref)
        return pltpu.async_copy(table_hbm.at[idx_ref], rows_ref, sem)

    def fire_write(rows_ref, off, sem):
        return pltpu.async_copy(rows_ref, out_hbm.at[pl.ds(off, CHUNK)], sem)

    def drain_gather(rows_ref, sem):
        pltpu.make_async_copy(table_hbm.at[pl.ds(0, CHUNK)], rows_ref, sem).wait()

    def drain_write(rows_ref, sem):
        pltpu.make_async_copy(rows_ref, out_hbm.at[pl.ds(base, CHUNK)], sem).wait()

    dA = fire_gather(idxA, rowsA, base, gA)
    dB = fire_gather(idxB, rowsB, base + CHUNK, gB)
    dA.wait()
    fire_write(rowsA, base, wA)
    dB.wait()
    fire_write(rowsB, base + CHUNK, wB)
    drain_write(rowsA, wA)
    fire_gather(idxA, rowsA, base + 2 * CHUNK, gA)

    def body(g, carry):
        offA = base + (2 * g) * CHUNK
        offB = offA + CHUNK
        drain_write(rowsB, wB)
        dBg = fire_gather(idxB, rowsB, offB, gB)
        drain_gather(rowsA, gA)
        fire_write(rowsA, offA, wA)
        dBg.wait()
        fire_write(rowsB, offB, wB)
        drain_write(rowsA, wA)
        fire_gather(idxA, rowsA, offA + 2 * CHUNK, gA)
        return carry

    lax.fori_loop(1, N_PAIRS - 1, body, 0)

    offA = base + (N_CHUNKS - 2) * CHUNK
    offB = offA + CHUNK
    drain_write(rowsB, wB)
    dBl = fire_gather(idxB, rowsB, offB, gB)
    drain_gather(rowsA, gA)
    fire_write(rowsA, offA, wA)
    dBl.wait()
    fire_write(rowsB, offB, wB)
    drain_write(rowsA, wA)
    drain_write(rowsB, wB)


def kernel(token_ids, emb):
    flat_ids = (token_ids.reshape(-1) * 2).astype(jnp.int32)
    emb_p = jnp.pad(emb, ((0, 0), (0, DIM))).reshape(2 * NUM_EMB, DIM)
    out = _gather_kernel(flat_ids, emb_p)
    return out.reshape(BATCH, HIST, DIM)
